# baseline (speedup 1.0000x reference)
import functools

import jax
import jax.numpy as jnp
from jax import lax
from jax.experimental import pallas as pl
from jax.experimental.pallas import tpu as pltpu
from jax.experimental.pallas import tpu_sc as plsc

D = 64
NC = 2
NS = 16
NW = NC * NS
BB = 128
V = 1000000
NTC = V // BB
VHALF = V // 2


def _repack_body(wt_hbm, t2_hbm, ins, outs, isems, osems):
    wid = lax.axis_index("s") * NC + lax.axis_index("c")
    iota = lax.iota(jnp.int32, 16)

    def start_in(tc, b):
        ts = pl.multiple_of(tc * BB, BB)
        pltpu.async_copy(wt_hbm.at[:, pl.ds(ts, BB)], ins[b], isems[b])

    def wait_in(b):
        pltpu.make_async_copy(wt_hbm.at[:, pl.ds(0, BB)], ins[b],
                              isems[b]).wait()

    def start_out(tc, b, nrows):
        pltpu.async_copy(outs[b].at[pl.ds(0, nrows)],
                         t2_hbm.at[pl.ds(tc * (BB // 2), nrows)], osems[b])

    def wait_out(b):
        pltpu.make_async_copy(outs[b].at[pl.ds(0, BB // 2)],
                              t2_hbm.at[pl.ds(0, BB // 2)], osems[b]).wait()

    def transpose(b):
        for m in range(D // 16):
            dvec = iota + 16 * m

            @plsc.parallel_loop(0, BB // 16, unroll=1)
            def _(tb):
                for k in range(16):
                    rot = (iota + k) & 15
                    tvec = rot + 16 * tb
                    jvec = (rot >> 1) + 8 * tb
                    cvec = ((rot & 1) << 6) + dvec
                    vals = plsc.load_gather(ins[b], [dvec, tvec])
                    plsc.store_scatter(outs[b], [jvec, cvec], vals)

    start_in(jnp.int32(wid), 0)

    def body(io, carry):
        for par in range(2):
            i = 2 * io + par
            tc = NW * i + wid

            @pl.when(tc < NTC)
            def _():
                @pl.when(tc + NW < NTC)
                def _():
                    start_in(tc + NW, 1 - par)

                wait_in(par)

                @pl.when(i >= 2)
                def _():
                    wait_out(par)

                transpose(par)
                start_out(tc, par, BB // 2)
        return carry

    lax.fori_loop(0, ((NTC + NW - 1) // NW + 1) // 2, body, 0)
    wait_out(0)
    wait_out(1)

    @pl.when(wid == 0)
    def _():
        start_in(jnp.int32(NTC), 0)
        wait_in(0)
        transpose(0)
        start_out(jnp.int32(NTC), 0, BB // 4)
        pltpu.make_async_copy(outs[0].at[pl.ds(0, BB // 4)],
                              t2_hbm.at[pl.ds(0, BB // 4)], osems[0]).wait()


def _emb_body(tids_hbm, t2_hbm, out_hbm, pid_v, cb_v, rows, outs,
              isem, gsems, wsems, *, seq):
    wid = lax.axis_index("s") * NC + lax.axis_index("c")
    b0 = wid * BB

    pltpu.async_copy(tids_hbm.at[:, pl.ds(b0, BB)], pid_v, isem).wait()

    def prep(s, carry):
        for g in range(BB // 16):
            v = pid_v[s, pl.ds(16 * g, 16)]
            cb_v[s, pl.ds(16 * g, 16)] = (v & 1) << 6
            pid_v[s, pl.ds(16 * g, 16)] = v >> 1
        return carry

    lax.fori_loop(0, seq, prep, 0)

    def gather(s, b):
        pltpu.async_copy(t2_hbm.at[pid_v.at[s]], rows[b], gsems[b])

    def wait_gather(b):
        pltpu.make_async_copy(t2_hbm.at[pid_v.at[0]], rows[b],
                              gsems[b]).wait()

    def writeback(s, b):
        pltpu.async_copy(outs[b], out_hbm.at[s, :, pl.ds(b0, BB)], wsems[b])

    def wait_writeback(b):
        pltpu.make_async_copy(outs[b], out_hbm.at[0, :, pl.ds(b0, BB)],
                              wsems[b]).wait()

    iota = lax.iota(jnp.int32, 16)

    def transpose(s, b):
        rv, ov = rows[b], outs[b]
        for g in range(BB // 16):
            b16 = iota + 16 * g
            cb16 = cb_v[s, pl.ds(16 * g, 16)]

            @plsc.parallel_loop(0, D // 16, unroll=1)
            def _(m):
                for k in range(16):
                    rot = ((iota + k) & 15) + 16 * m
                    vals = plsc.load_gather(rv, [b16, cb16 + rot])
                    plsc.store_scatter(ov, [rot, b16], vals)

    gather(0, 0)
    gather(1, 1)

    def sbody(so, carry):
        for par in range(3):
            s = 3 * so + par

            @pl.when(s + 2 < seq)
            def _():
                gather(s + 2, (par + 2) % 3)

            wait_gather(par)

            @pl.when(s >= 3)
            def _():
                wait_writeback(par)

            transpose(s, par)
            writeback(s, par)
        return carry

    lax.fori_loop(0, seq // 3, sbody, 0)
    for s in range(3 * (seq // 3), seq):
        par = s % 3
        wait_gather(par)
        wait_writeback(par)
        transpose(s, par)
        writeback(s, par)
    wait_writeback(0)
    wait_writeback(1)
    wait_writeback(2)


@functools.partial(jax.jit, static_argnames=("seq",))
def _emb(tids, wt, seq):
    mesh = plsc.VectorSubcoreMesh(core_axis_name="c", subcore_axis_name="s")
    table2 = pl.kernel(
        _repack_body,
        mesh=mesh,
        out_type=jax.ShapeDtypeStruct((VHALF, 2 * D), jnp.float32),
        scratch_types=[
            [pltpu.VMEM((D, BB), jnp.float32) for _ in range(2)],
            [pltpu.VMEM((D, 2 * D), jnp.float32) for _ in range(2)],
            [pltpu.SemaphoreType.DMA for _ in range(2)],
            [pltpu.SemaphoreType.DMA for _ in range(2)],
        ],
        compiler_params=pltpu.CompilerParams(needs_layout_passes=False),
    )(wt)
    body = functools.partial(_emb_body, seq=seq)
    return pl.kernel(
        body,
        mesh=mesh,
        out_type=jax.ShapeDtypeStruct((seq, D, NW * BB), jnp.float32),
        scratch_types=[
            pltpu.VMEM((seq, BB), jnp.int32),
            pltpu.VMEM((seq, BB), jnp.int32),
            [pltpu.VMEM((BB, 2 * D), jnp.float32) for _ in range(3)],
            [pltpu.VMEM((D, BB), jnp.float32) for _ in range(3)],
            pltpu.SemaphoreType.DMA,
            [pltpu.SemaphoreType.DMA for _ in range(3)],
            [pltpu.SemaphoreType.DMA for _ in range(3)],
        ],
        compiler_params=pltpu.CompilerParams(needs_layout_passes=False),
    )(tids, table2)


def kernel(token_ids, W):
    b, s = token_ids.shape
    tids = token_ids.astype(jnp.int32).T
    wt = W.T
    out_phys = _emb(tids, wt, s)
    return jnp.transpose(out_phys, (2, 0, 1))


# speedup vs baseline: 1.2443x; 1.2443x over previous
import functools

import jax
import jax.numpy as jnp
from jax import lax
from jax.experimental import pallas as pl
from jax.experimental.pallas import tpu as pltpu
from jax.experimental.pallas import tpu_sc as plsc

D = 64
NC = 2
NS = 16
NW = NC * NS
BB = 128
V = 1000000
NTC = V // BB
VHALF = V // 2


def _repack_body(wt_hbm, t2_hbm, ins, outs, isems, osems):
    wid = lax.axis_index("s") * NC + lax.axis_index("c")
    iota = lax.iota(jnp.int32, 16)

    def start_in(tc, b):
        ts = pl.multiple_of(tc * BB, BB)
        pltpu.async_copy(wt_hbm.at[:, pl.ds(ts, BB)], ins[b], isems[b])

    def wait_in(b):
        pltpu.make_async_copy(wt_hbm.at[:, pl.ds(0, BB)], ins[b],
                              isems[b]).wait()

    def start_out(tc, b, nrows):
        pltpu.async_copy(outs[b].at[pl.ds(0, nrows)],
                         t2_hbm.at[pl.ds(tc * (BB // 2), nrows)], osems[b])

    def wait_out(b):
        pltpu.make_async_copy(outs[b].at[pl.ds(0, BB // 2)],
                              t2_hbm.at[pl.ds(0, BB // 2)], osems[b]).wait()

    def transpose(b):
        for m in range(D // 16):
            dvec = iota + 16 * m

            @plsc.parallel_loop(0, BB // 16, unroll=1)
            def _(tb):
                for k in range(16):
                    rot = (iota + k) & 15
                    tvec = rot + 16 * tb
                    jvec = (rot >> 1) + 8 * tb
                    cvec = ((rot & 1) << 6) + dvec
                    vals = plsc.load_gather(ins[b], [dvec, tvec])
                    plsc.store_scatter(outs[b], [jvec, cvec], vals)

    start_in(jnp.int32(wid), 0)

    def body(io, carry):
        for par in range(2):
            i = 2 * io + par
            tc = NW * i + wid

            @pl.when(tc < NTC)
            def _():
                @pl.when(tc + NW < NTC)
                def _():
                    start_in(tc + NW, 1 - par)

                wait_in(par)

                @pl.when(i >= 2)
                def _():
                    wait_out(par)

                transpose(par)
                start_out(tc, par, BB // 2)
        return carry

    lax.fori_loop(0, ((NTC + NW - 1) // NW + 1) // 2, body, 0)
    wait_out(0)
    wait_out(1)

    @pl.when(wid == 0)
    def _():
        start_in(jnp.int32(NTC), 0)
        wait_in(0)
        transpose(0)
        start_out(jnp.int32(NTC), 0, BB // 4)
        pltpu.make_async_copy(outs[0].at[pl.ds(0, BB // 4)],
                              t2_hbm.at[pl.ds(0, BB // 4)], osems[0]).wait()


def _emb_body(tids_hbm, t2_hbm, out_hbm, idx_v, pid_v, cb_v, rows, outs,
              isem, gsems, wsems, *, seq):
    wid = lax.axis_index("s") * NC + lax.axis_index("c")
    b0 = wid * BB

    pltpu.async_copy(tids_hbm.at[:, pl.ds(b0, BB)], idx_v, isem).wait()

    def prep(s, carry):
        for g in range(BB // 16):
            v = idx_v[s, pl.ds(16 * g, 16)]
            pid_v[s, pl.ds(16 * g, 16)] = v >> 1
            cb_v[s, pl.ds(16 * g, 16)] = (v & 1) << 6
        return carry

    lax.fori_loop(0, seq, prep, 0)

    def gather(s, b):
        pltpu.async_copy(t2_hbm.at[pid_v.at[s]], rows[b], gsems[b])

    def wait_gather(b):
        pltpu.make_async_copy(t2_hbm.at[pid_v.at[0]], rows[b],
                              gsems[b]).wait()

    def writeback(s, b):
        pltpu.async_copy(outs[b], out_hbm.at[s, :, pl.ds(b0, BB)], wsems[b])

    def wait_writeback(b):
        pltpu.make_async_copy(outs[b], out_hbm.at[0, :, pl.ds(b0, BB)],
                              wsems[b]).wait()

    iota = lax.iota(jnp.int32, 16)

    def transpose(s, b):
        rv, ov = rows[b], outs[b]
        for g in range(BB // 16):
            b16 = iota + 16 * g
            cb16 = cb_v[s, pl.ds(16 * g, 16)]

            @plsc.parallel_loop(0, D // 16, unroll=2)
            def _(m):
                for k in range(16):
                    rot = ((iota + k) & 15) + 16 * m
                    vals = plsc.load_gather(rv, [b16, cb16 + rot])
                    plsc.store_scatter(ov, [rot, b16], vals)

    gather(0, 0)

    def sbody(so, carry):
        for par in range(2):
            s = 2 * so + par

            @pl.when(s + 1 < seq)
            def _():
                gather(s + 1, 1 - par)

            wait_gather(par)

            @pl.when(s >= 2)
            def _():
                wait_writeback(par)

            transpose(s, par)
            writeback(s, par)
        return carry

    lax.fori_loop(0, seq // 2, sbody, 0)
    wait_writeback(0)
    wait_writeback(1)


@functools.partial(jax.jit, static_argnames=("seq",))
def _emb(tids, wt, seq):
    mesh = plsc.VectorSubcoreMesh(core_axis_name="c", subcore_axis_name="s")
    table2 = pl.kernel(
        _repack_body,
        mesh=mesh,
        out_type=jax.ShapeDtypeStruct((VHALF, 2 * D), jnp.float32),
        scratch_types=[
            [pltpu.VMEM((D, BB), jnp.float32) for _ in range(2)],
            [pltpu.VMEM((D, 2 * D), jnp.float32) for _ in range(2)],
            [pltpu.SemaphoreType.DMA for _ in range(2)],
            [pltpu.SemaphoreType.DMA for _ in range(2)],
        ],
        compiler_params=pltpu.CompilerParams(needs_layout_passes=False),
    )(wt)
    body = functools.partial(_emb_body, seq=seq)
    return pl.kernel(
        body,
        mesh=mesh,
        out_type=jax.ShapeDtypeStruct((seq, D, NW * BB), jnp.float32),
        scratch_types=[
            pltpu.VMEM((seq, BB), jnp.int32),
            pltpu.VMEM((seq, BB), jnp.int32),
            pltpu.VMEM((seq, BB), jnp.int32),
            [pltpu.VMEM((BB, 2 * D), jnp.float32) for _ in range(2)],
            [pltpu.VMEM((D, BB), jnp.float32) for _ in range(2)],
            pltpu.SemaphoreType.DMA,
            [pltpu.SemaphoreType.DMA for _ in range(2)],
            [pltpu.SemaphoreType.DMA for _ in range(2)],
        ],
        compiler_params=pltpu.CompilerParams(needs_layout_passes=False),
    )(tids, table2)


def kernel(token_ids, W):
    b, s = token_ids.shape
    tids = token_ids.astype(jnp.int32).T
    wt = W.T
    out_phys = _emb(tids, wt, s)
    return jnp.transpose(out_phys, (2, 0, 1))
